# Initial kernel scaffold; baseline (speedup 1.0000x reference)
import functools

import jax
import jax.numpy as jnp
from jax import lax
from jax.experimental import pallas as pl
from jax.experimental.pallas import tpu as pltpu
from jax.experimental.pallas import tpu_sc as plsc

N_USERS = 25000
N_NODES = 50000
DIM = 64
N_EDGES = 800000
HALF = 25000
ACC_ROWS = 25600
DUMMY = HALF
NC, NS = 2, 16
EPW = N_EDGES // NS
OUTER = 2000
SUB = 80
RPW = ACC_ROWS // NS
OCH = 200
NOCH = HALF // OCH
PAIRS = 4096
PPW = PAIRS // (NC * NS)

_mesh = plsc.VectorSubcoreMesh(core_axis_name="c", subcore_axis_name="s")

_BCAST_DNUMS = lax.GatherDimensionNumbers(
    offset_dims=(), collapsed_slice_dims=(0,), start_index_map=(0,))


def _bcast(v, j):
    idx = jnp.full((16,), j, dtype=jnp.int32)
    return lax.gather(v, idx[:, None], _BCAST_DNUMS, (1,),
                      mode=lax.GatherScatterMode.PROMISE_IN_BOUNDS)


@functools.partial(
    pl.kernel,
    out_type=jax.ShapeDtypeStruct((N_NODES, DIM), jnp.float32),
    mesh=_mesh,
    scratch_types=[
        pltpu.VMEM_SHARED((ACC_ROWS, DIM), jnp.float32),
        pltpu.VMEM((OUTER,), jnp.int32),
        pltpu.VMEM((OUTER,), jnp.float32),
        pltpu.VMEM((SUB,), jnp.int32),
        pltpu.VMEM((SUB,), jnp.int32),
        pltpu.VMEM((SUB, DIM), jnp.float32),
        pltpu.VMEM((OCH, DIM), jnp.float32),
        pltpu.VMEM((OCH, DIM), jnp.float32),
        pltpu.SemaphoreType.DMA,
    ],
)
def _spmm(emb_in, src_hbm, dst_hbm, val_hbm, zeros_hbm, out,
          acc, dstst, valst, src_v, idx_v, rows, inbuf, accbuf, sem):
    c = lax.axis_index("c")
    s = lax.axis_index("s")
    pltpu.sync_copy(zeros_hbm, acc.at[pl.ds(s * RPW, RPW)])
    plsc.subcore_barrier()

    base_w = s * EPW

    def outer_body(o, _):
        ob = base_w + o * OUTER
        pltpu.sync_copy(dst_hbm.at[pl.ds(ob, OUTER)], dstst)
        pltpu.sync_copy(val_hbm.at[pl.ds(ob, OUTER)], valst)

        def sub_body(u, _):
            sb = ob + u * SUB
            pltpu.sync_copy(src_hbm.at[pl.ds(sb, SUB)], src_v)
            pltpu.async_copy(emb_in.at[src_v], rows, sem).wait()
            off = u * SUB

            def grp(q, _):
                go = off + q * 16
                dv = dstst[pl.ds(go, 16)]
                rel = dv - c * HALF
                ok = (rel >= 0) & (rel < HALF)
                idx_v[pl.ds(q * 16, 16)] = jnp.where(ok, rel, DUMMY)
                vv = valst[pl.ds(go, 16)]
                for j in range(16):
                    vb = _bcast(vv, j)
                    r = q * 16 + j
                    for k in range(4):
                        sl = pl.ds(k * 16, 16)
                        rows[r, sl] = rows[r, sl] * vb
                return 0

            lax.fori_loop(0, SUB // 16, grp, 0)
            pltpu.sync_copy(rows, acc.at[idx_v], add=True)
            return 0

        lax.fori_loop(0, OUTER // SUB, sub_body, 0)
        return 0

    lax.fori_loop(0, EPW // OUTER, outer_body, 0)
    plsc.subcore_barrier()

    nch = (NOCH - s + NS - 1) // NS

    def och_body(t, _):
        ch = s + t * NS
        rel0 = ch * OCH
        row0 = c * HALF + rel0
        pltpu.sync_copy(emb_in.at[pl.ds(row0, OCH)], inbuf)
        pltpu.sync_copy(acc.at[pl.ds(rel0, OCH)], accbuf)

        def rowb(r, _):
            for k in range(4):
                sl = pl.ds(k * 16, 16)
                accbuf[r, sl] = 0.2 * inbuf[r, sl] + 0.8 * accbuf[r, sl]
            return 0

        lax.fori_loop(0, OCH, rowb, 0)
        pltpu.sync_copy(accbuf, out.at[pl.ds(row0, OCH)])
        return 0

    lax.fori_loop(0, nch, och_body, 0)


@functools.partial(
    pl.kernel,
    out_type=jax.ShapeDtypeStruct((PAIRS,), jnp.float32),
    mesh=_mesh,
    scratch_types=[
        pltpu.VMEM((PPW,), jnp.int32),
        pltpu.VMEM((PPW,), jnp.int32),
        pltpu.VMEM((PPW, DIM), jnp.float32),
        pltpu.VMEM((PPW, DIM), jnp.float32),
        pltpu.VMEM((PPW,), jnp.float32),
        pltpu.SemaphoreType.DMA,
    ],
)
def _pairdot(e0, e1, e2, e3, users, items, out,
             uidx, iidx, ubuf, ibuf, gbuf, sem):
    c = lax.axis_index("c")
    s = lax.axis_index("s")
    w = s * NC + c
    pb = w * PPW
    pltpu.sync_copy(users.at[pl.ds(pb, PPW)], uidx)
    pltpu.sync_copy(items.at[pl.ds(pb, PPW)], iidx)

    def addoff(q, _):
        sl = pl.ds(q * 16, 16)
        iidx[sl] = iidx[sl] + N_USERS
        return 0

    lax.fori_loop(0, PPW // 16, addoff, 0)

    pltpu.async_copy(e0.at[uidx], ubuf, sem).wait()
    pltpu.async_copy(e1.at[uidx], ubuf, sem, add=True).wait()
    pltpu.async_copy(e2.at[uidx], ubuf, sem, add=True).wait()
    pltpu.async_copy(e3.at[uidx], ubuf, sem, add=True).wait()
    pltpu.async_copy(e0.at[iidx], ibuf, sem).wait()
    pltpu.async_copy(e1.at[iidx], ibuf, sem, add=True).wait()
    pltpu.async_copy(e2.at[iidx], ibuf, sem, add=True).wait()
    pltpu.async_copy(e3.at[iidx], ibuf, sem, add=True).wait()

    lanes = lax.broadcasted_iota(jnp.int32, (16,), 0)

    def gloop(g, _):
        rows16 = lanes + g * 16

        def dloop(d, a):
            dd = jnp.full((16,), 0, jnp.int32) + d
            uu = plsc.load_gather(ubuf, [rows16, dd])
            ii = plsc.load_gather(ibuf, [rows16, dd])
            return a + uu * ii

        a = lax.fori_loop(0, DIM, dloop, jnp.zeros((16,), jnp.float32))
        gbuf[pl.ds(g * 16, 16)] = a * 0.0625
        return 0

    lax.fori_loop(0, PPW // 16, gloop, 0)
    pltpu.sync_copy(gbuf, out.at[pl.ds(pb, PPW)])


def kernel(users, items, edge_index, edge_vals, user_emb, item_emb):
    all0 = jnp.concatenate([user_emb, item_emb], axis=0)
    dst = edge_index[0]
    src = edge_index[1]
    zeros = jnp.zeros((RPW, DIM), jnp.float32)
    e1 = _spmm(all0, src, dst, edge_vals, zeros)
    e2 = _spmm(e1, src, dst, edge_vals, zeros)
    e3 = _spmm(e2, src, dst, edge_vals, zeros)
    return _pairdot(all0, e1, e2, e3, users, items)


# speedup vs baseline: 2.1103x; 2.1103x over previous
import functools

import jax
import jax.numpy as jnp
from jax import lax
from jax.experimental import pallas as pl
from jax.experimental.pallas import tpu as pltpu
from jax.experimental.pallas import tpu_sc as plsc

N_USERS = 25000
N_NODES = 50000
DIM = 64
N_EDGES = 800000
HALF = 25000
ACC_ROWS = 25008
DUMMY = HALF
NC, NS = 2, 16
EPW = N_EDGES // NS
OUTER = 2000
SUB = 80
RPW = ACC_ROWS // NS
OCH = 100
NOCH = HALF // OCH
PAIRS = 4096
PPW = PAIRS // (NC * NS)

_mesh = plsc.VectorSubcoreMesh(core_axis_name="c", subcore_axis_name="s")

_BCAST_DNUMS = lax.GatherDimensionNumbers(
    offset_dims=(), collapsed_slice_dims=(0,), start_index_map=(0,))


def _bcast(v, j):
    idx = jnp.full((16,), j, dtype=jnp.int32)
    return lax.gather(v, idx[:, None], _BCAST_DNUMS, (1,),
                      mode=lax.GatherScatterMode.PROMISE_IN_BOUNDS)


@functools.partial(
    pl.kernel,
    out_type=jax.ShapeDtypeStruct((N_NODES, DIM), jnp.float32),
    mesh=_mesh,
    compiler_params=pltpu.CompilerParams(use_tc_tiling_on_sc=False, needs_layout_passes=False),
    scratch_types=[
        pltpu.VMEM_SHARED((ACC_ROWS, DIM), jnp.float32),
        pltpu.VMEM((OUTER,), jnp.int32),
        pltpu.VMEM((OUTER,), jnp.float32),
        pltpu.VMEM((SUB,), jnp.int32),
        pltpu.VMEM((SUB,), jnp.int32),
        pltpu.VMEM((SUB, DIM), jnp.float32),
        pltpu.VMEM((OCH, DIM), jnp.float32),
        pltpu.VMEM((OCH, DIM), jnp.float32),
        pltpu.SemaphoreType.DMA,
    ],
)
def _spmm(emb_in, src_hbm, dst_hbm, val_hbm, zeros_hbm, out,
          acc, dstst, valst, src_v, idx_v, rows, inbuf, accbuf, sem):
    c = lax.axis_index("c")
    s = lax.axis_index("s")
    pltpu.sync_copy(zeros_hbm, acc.at[pl.ds(s * RPW, RPW)])
    plsc.subcore_barrier()

    base_w = s * EPW

    def outer_body(o, _):
        ob = base_w + o * OUTER
        pltpu.sync_copy(dst_hbm.at[pl.ds(ob, OUTER)], dstst)
        pltpu.sync_copy(val_hbm.at[pl.ds(ob, OUTER)], valst)

        def sub_body(u, _):
            sb = ob + u * SUB
            pltpu.sync_copy(src_hbm.at[pl.ds(sb, SUB)], src_v)
            pltpu.async_copy(emb_in.at[src_v], rows, sem).wait()
            off = u * SUB

            def grp(q, _):
                go = off + q * 16
                dv = dstst[pl.ds(go, 16)]
                rel = dv - c * HALF
                ok = (rel >= 0) & (rel < HALF)
                idx_v[pl.ds(q * 16, 16)] = jnp.where(ok, rel, DUMMY)
                vv = valst[pl.ds(go, 16)]
                for j in range(16):
                    vb = _bcast(vv, j)
                    r = q * 16 + j
                    for k in range(4):
                        sl = pl.ds(k * 16, 16)
                        rows[r, sl] = rows[r, sl] * vb
                return 0

            lax.fori_loop(0, SUB // 16, grp, 0)
            pltpu.sync_copy(rows, acc.at[idx_v], add=True)
            return 0

        lax.fori_loop(0, OUTER // SUB, sub_body, 0)
        return 0

    lax.fori_loop(0, EPW // OUTER, outer_body, 0)
    plsc.subcore_barrier()

    nch = (NOCH - s + NS - 1) // NS

    def och_body(t, _):
        ch = s + t * NS
        rel0 = ch * OCH
        row0 = c * HALF + rel0
        pltpu.sync_copy(emb_in.at[pl.ds(row0, OCH)], inbuf)
        pltpu.sync_copy(acc.at[pl.ds(rel0, OCH)], accbuf)

        def rowb(r, _):
            for k in range(4):
                sl = pl.ds(k * 16, 16)
                accbuf[r, sl] = 0.2 * inbuf[r, sl] + 0.8 * accbuf[r, sl]
            return 0

        lax.fori_loop(0, OCH, rowb, 0)
        pltpu.sync_copy(accbuf, out.at[pl.ds(row0, OCH)])
        return 0

    lax.fori_loop(0, nch, och_body, 0)


@functools.partial(
    pl.kernel,
    out_type=jax.ShapeDtypeStruct((PAIRS,), jnp.float32),
    mesh=_mesh,
    compiler_params=pltpu.CompilerParams(use_tc_tiling_on_sc=False, needs_layout_passes=False),
    scratch_types=[
        pltpu.VMEM((PPW,), jnp.int32),
        pltpu.VMEM((PPW,), jnp.int32),
        pltpu.VMEM((PPW, DIM), jnp.float32),
        pltpu.VMEM((PPW, DIM), jnp.float32),
        pltpu.VMEM((PPW,), jnp.float32),
        pltpu.SemaphoreType.DMA,
    ],
)
def _pairdot(e0, e1, e2, e3, users, items, out,
             uidx, iidx, ubuf, ibuf, gbuf, sem):
    c = lax.axis_index("c")
    s = lax.axis_index("s")
    w = s * NC + c
    pb = w * PPW
    pltpu.sync_copy(users.at[pl.ds(pb, PPW)], uidx)
    pltpu.sync_copy(items.at[pl.ds(pb, PPW)], iidx)

    def addoff(q, _):
        sl = pl.ds(q * 16, 16)
        iidx[sl] = iidx[sl] + N_USERS
        return 0

    lax.fori_loop(0, PPW // 16, addoff, 0)

    pltpu.async_copy(e0.at[uidx], ubuf, sem).wait()
    pltpu.async_copy(e1.at[uidx], ubuf, sem, add=True).wait()
    pltpu.async_copy(e2.at[uidx], ubuf, sem, add=True).wait()
    pltpu.async_copy(e3.at[uidx], ubuf, sem, add=True).wait()
    pltpu.async_copy(e0.at[iidx], ibuf, sem).wait()
    pltpu.async_copy(e1.at[iidx], ibuf, sem, add=True).wait()
    pltpu.async_copy(e2.at[iidx], ibuf, sem, add=True).wait()
    pltpu.async_copy(e3.at[iidx], ibuf, sem, add=True).wait()

    lanes = lax.broadcasted_iota(jnp.int32, (16,), 0)

    def gloop(g, _):
        rows16 = lanes + g * 16

        def dloop(d, a):
            dd = jnp.full((16,), 0, jnp.int32) + d
            uu = plsc.load_gather(ubuf, [rows16, dd])
            ii = plsc.load_gather(ibuf, [rows16, dd])
            return a + uu * ii

        a = lax.fori_loop(0, DIM, dloop, jnp.zeros((16,), jnp.float32))
        gbuf[pl.ds(g * 16, 16)] = a * 0.0625
        return 0

    lax.fori_loop(0, PPW // 16, gloop, 0)
    pltpu.sync_copy(gbuf, out.at[pl.ds(pb, PPW)])


def kernel(users, items, edge_index, edge_vals, user_emb, item_emb):
    all0 = jnp.concatenate([user_emb, item_emb], axis=0)
    dst = edge_index[0]
    src = edge_index[1]
    zeros = jnp.zeros((RPW, DIM), jnp.float32)
    e1 = _spmm(all0, src, dst, edge_vals, zeros)
    e2 = _spmm(e1, src, dst, edge_vals, zeros)
    e3 = _spmm(e2, src, dst, edge_vals, zeros)
    return _pairdot(all0, e1, e2, e3, users, items)


# speedup vs baseline: 5.8038x; 2.7502x over previous
import functools

import jax
import jax.numpy as jnp
from jax import lax
from jax.experimental import pallas as pl
from jax.experimental.pallas import tpu as pltpu
from jax.experimental.pallas import tpu_sc as plsc

N_USERS = 25000
N_NODES = 50000
DIM = 64
N_EDGES = 800000
HALF = 25000
ACC_ROWS = 25008
DUMMY = HALF
NC, NS = 2, 16
EPW = N_EDGES // NS
OUTER = 2000
SUB = 80
RPW = ACC_ROWS // NS
OCH = 50
NSUB = EPW // SUB
SPO = OUTER // SUB
NOCH = HALF // OCH
PAIRS = 4096
PPW = PAIRS // (NC * NS)

_mesh = plsc.VectorSubcoreMesh(core_axis_name="c", subcore_axis_name="s")

_BCAST_DNUMS = lax.GatherDimensionNumbers(
    offset_dims=(), collapsed_slice_dims=(0,), start_index_map=(0,))


def _bcast(v, j):
    idx = jnp.full((16,), j, dtype=jnp.int32)
    return lax.gather(v, idx[:, None], _BCAST_DNUMS, (1,),
                      mode=lax.GatherScatterMode.PROMISE_IN_BOUNDS)


@functools.partial(
    pl.kernel,
    out_type=jax.ShapeDtypeStruct((N_NODES, DIM), jnp.float32),
    mesh=_mesh,
    compiler_params=pltpu.CompilerParams(use_tc_tiling_on_sc=False, needs_layout_passes=False),
    scratch_types=[
        pltpu.VMEM_SHARED((ACC_ROWS, DIM), jnp.float32),
        pltpu.VMEM((OUTER,), jnp.int32),
        pltpu.VMEM((OUTER,), jnp.int32),
        pltpu.VMEM((OUTER,), jnp.float32),
        pltpu.VMEM((2, SUB), jnp.int32),
        pltpu.VMEM((2, SUB, DIM), jnp.float32),
        pltpu.VMEM((OCH, DIM), jnp.float32),
        pltpu.VMEM((OCH, DIM), jnp.float32),
        pltpu.SemaphoreType.DMA,
        pltpu.SemaphoreType.DMA,
    ],
)
def _spmm(emb_in, src_hbm, dst_hbm, val_hbm, zeros_hbm, out,
          acc, srcst, dstst, valst, idx2, rows2, inbuf, accbuf, gsem, ssem):
    c = lax.axis_index("c")
    s = lax.axis_index("s")
    pltpu.sync_copy(zeros_hbm, acc.at[pl.ds(s * RPW, RPW)])
    plsc.subcore_barrier()

    base_w = s * EPW

    def stage(o):
        ob = base_w + o * OUTER
        pltpu.sync_copy(src_hbm.at[pl.ds(ob, OUTER)], srcst)
        pltpu.sync_copy(dst_hbm.at[pl.ds(ob, OUTER)], dstst)
        pltpu.sync_copy(val_hbm.at[pl.ds(ob, OUTER)], valst)

    def issue_gather(u, b):
        w0 = lax.rem(u, SPO) * SUB
        pltpu.async_copy(emb_in.at[srcst.at[pl.ds(w0, SUB)]], rows2.at[b], gsem)

    def wait_gather(b):
        pltpu.make_async_copy(emb_in.at[srcst.at[pl.ds(0, SUB)]],
                              rows2.at[b], gsem).wait()

    def wait_scatter(b):
        pltpu.make_async_copy(rows2.at[b], acc.at[idx2.at[b]], ssem).wait()

    stage(0)
    issue_gather(0, 0)

    def body(u, _):
        b = lax.rem(u, 2)
        nb = 1 - b
        wait_gather(b)
        boundary = lax.rem(u + 1, SPO) == 0
        notlast = u + 1 < NSUB

        @pl.when(u > 0)
        def _():
            wait_scatter(nb)

        @pl.when(notlast & jnp.logical_not(boundary))
        def _():
            issue_gather(u + 1, nb)

        w0 = lax.rem(u, SPO) * SUB

        def grp(q, _):
            go = w0 + q * 16
            dv = dstst[pl.ds(go, 16)]
            rel = dv - c * HALF
            ok = (rel >= 0) & (rel < HALF)
            idx2[b, pl.ds(q * 16, 16)] = jnp.where(ok, rel, DUMMY)
            vv = valst[pl.ds(go, 16)]
            for j in range(16):
                vb = _bcast(vv, j)
                r = q * 16 + j
                for k in range(4):
                    sl = pl.ds(k * 16, 16)
                    rows2[b, r, sl] = rows2[b, r, sl] * vb
            return 0

        lax.fori_loop(0, SUB // 16, grp, 0)

        @pl.when(boundary & notlast)
        def _():
            stage((u + 1) // SPO)
            issue_gather(u + 1, nb)

        pltpu.async_copy(rows2.at[b], acc.at[idx2.at[b]], ssem, add=True)
        return 0

    lax.fori_loop(0, NSUB, body, 0)
    wait_scatter((NSUB - 1) % 2)
    plsc.subcore_barrier()

    nch = (NOCH - s + NS - 1) // NS

    def och_body(t, _):
        ch = s + t * NS
        rel0 = ch * OCH
        row0 = c * HALF + rel0
        pltpu.sync_copy(emb_in.at[pl.ds(row0, OCH)], inbuf)
        pltpu.sync_copy(acc.at[pl.ds(rel0, OCH)], accbuf)

        def rowb(r, _):
            for k in range(4):
                sl = pl.ds(k * 16, 16)
                accbuf[r, sl] = 0.2 * inbuf[r, sl] + 0.8 * accbuf[r, sl]
            return 0

        lax.fori_loop(0, OCH, rowb, 0)
        pltpu.sync_copy(accbuf, out.at[pl.ds(row0, OCH)])
        return 0

    lax.fori_loop(0, nch, och_body, 0)


@functools.partial(
    pl.kernel,
    out_type=jax.ShapeDtypeStruct((PAIRS,), jnp.float32),
    mesh=_mesh,
    compiler_params=pltpu.CompilerParams(use_tc_tiling_on_sc=False, needs_layout_passes=False),
    scratch_types=[
        pltpu.VMEM((PPW,), jnp.int32),
        pltpu.VMEM((PPW,), jnp.int32),
        pltpu.VMEM((PPW, DIM), jnp.float32),
        pltpu.VMEM((PPW, DIM), jnp.float32),
        pltpu.VMEM((PPW,), jnp.float32),
        pltpu.SemaphoreType.DMA,
    ],
)
def _pairdot(e0, e1, e2, e3, users, items, out,
             uidx, iidx, ubuf, ibuf, gbuf, sem):
    c = lax.axis_index("c")
    s = lax.axis_index("s")
    w = s * NC + c
    pb = w * PPW
    pltpu.sync_copy(users.at[pl.ds(pb, PPW)], uidx)
    pltpu.sync_copy(items.at[pl.ds(pb, PPW)], iidx)

    def addoff(q, _):
        sl = pl.ds(q * 16, 16)
        iidx[sl] = iidx[sl] + N_USERS
        return 0

    lax.fori_loop(0, PPW // 16, addoff, 0)

    pltpu.async_copy(e0.at[uidx], ubuf, sem).wait()
    pltpu.async_copy(e1.at[uidx], ubuf, sem, add=True).wait()
    pltpu.async_copy(e2.at[uidx], ubuf, sem, add=True).wait()
    pltpu.async_copy(e3.at[uidx], ubuf, sem, add=True).wait()
    pltpu.async_copy(e0.at[iidx], ibuf, sem).wait()
    pltpu.async_copy(e1.at[iidx], ibuf, sem, add=True).wait()
    pltpu.async_copy(e2.at[iidx], ibuf, sem, add=True).wait()
    pltpu.async_copy(e3.at[iidx], ibuf, sem, add=True).wait()

    lanes = lax.broadcasted_iota(jnp.int32, (16,), 0)

    def gloop(g, _):
        rows16 = lanes + g * 16

        def dloop(d, a):
            dd = jnp.full((16,), 0, jnp.int32) + d
            uu = plsc.load_gather(ubuf, [rows16, dd])
            ii = plsc.load_gather(ibuf, [rows16, dd])
            return a + uu * ii

        a = lax.fori_loop(0, DIM, dloop, jnp.zeros((16,), jnp.float32))
        gbuf[pl.ds(g * 16, 16)] = a * 0.0625
        return 0

    lax.fori_loop(0, PPW // 16, gloop, 0)
    pltpu.sync_copy(gbuf, out.at[pl.ds(pb, PPW)])


def kernel(users, items, edge_index, edge_vals, user_emb, item_emb):
    all0 = jnp.concatenate([user_emb, item_emb], axis=0)
    dst = edge_index[0]
    src = edge_index[1]
    zeros = jnp.zeros((RPW, DIM), jnp.float32)
    e1 = _spmm(all0, src, dst, edge_vals, zeros)
    e2 = _spmm(e1, src, dst, edge_vals, zeros)
    e3 = _spmm(e2, src, dst, edge_vals, zeros)
    return _pairdot(all0, e1, e2, e3, users, items)
